# Initial kernel scaffold; baseline (speedup 1.0000x reference)
import jax
import jax.numpy as jnp
from jax.experimental import pallas as pl


def kernel(input_ids, token_type_ids, task_type_ids, word_emb, pos_emb, tok_emb, task_emb, gamma, beta):
    raise NotImplementedError("write your pallas kernel here")


# speedup vs baseline: 5.0124x; 5.0124x over previous
import functools

import jax
import jax.numpy as jnp
from jax import lax
from jax.experimental import pallas as pl
from jax.experimental.pallas import tpu as pltpu
from jax.experimental.pallas import tpu_sc as plsc

HID = 128
EPS = 1e-12
CHUNK = 128


@functools.lru_cache(maxsize=None)
def _make_sc_gather(n_tokens: int):
    info = plsc.get_sparse_core_info()
    nc, ns = info.num_cores, info.num_subcores
    nw = nc * ns
    per_w = n_tokens // nw
    iters = per_w // CHUNK
    mesh = plsc.VectorSubcoreMesh(core_axis_name="c", subcore_axis_name="s")

    @functools.partial(
        pl.kernel,
        out_type=jax.ShapeDtypeStruct((n_tokens, HID), jnp.float32),
        mesh=mesh,
        scratch_types=[
            pltpu.VMEM((CHUNK,), jnp.int32),
            pltpu.VMEM((CHUNK, HID), jnp.float32),
            pltpu.SemaphoreType.DMA,
        ],
    )
    def gather(table_hbm, ids_hbm, out_hbm, idx_v, rows_v, sem):
        wid = lax.axis_index("s") * nc + lax.axis_index("c")
        base = wid * per_w

        def body(i, carry):
            off = base + i * CHUNK
            pltpu.sync_copy(ids_hbm.at[pl.ds(off, CHUNK)], idx_v)
            pltpu.async_copy(table_hbm.at[idx_v], rows_v, sem).wait()
            pltpu.sync_copy(rows_v, out_hbm.at[pl.ds(off, CHUNK)])
            return carry

        lax.fori_loop(0, iters, body, 0)

    return gather


def _ln_body(gath_ref, tt_ref, task_ref, pos_ref, tok_ref, taskemb_ref,
             gamma_ref, beta_ref, out_ref):
    s = gath_ref.shape[1]
    e = gath_ref[0] + pos_ref[...]
    tt = tt_ref[0, 0, :].astype(jnp.float32)[:, None]
    e = e + tok_ref[0, :][None, :] + tt * (tok_ref[1, :] - tok_ref[0, :])[None, :]
    task = task_ref[0, 0, :][:, None]
    oh = (task == lax.broadcasted_iota(jnp.int32, (s, 16), 1)).astype(jnp.float32)
    e = e + jnp.dot(oh, taskemb_ref[...], preferred_element_type=jnp.float32)
    mu = jnp.mean(e, axis=-1, keepdims=True)
    var = jnp.mean((e - mu) ** 2, axis=-1, keepdims=True)
    out = (e - mu) * lax.rsqrt(var + EPS) * gamma_ref[...] + beta_ref[...]
    out_ref[...] = out[None]


def kernel(input_ids, token_type_ids, task_type_ids, word_emb, pos_emb,
           tok_emb, task_emb, gamma, beta):
    b, s = input_ids.shape
    n = b * s
    ids = input_ids.reshape(n).astype(jnp.int32)
    gathered = _make_sc_gather(n)(word_emb, ids)

    tt3 = token_type_ids.reshape(b, 1, s).astype(jnp.int32)
    task3 = task_type_ids.reshape(b, 1, s).astype(jnp.int32)
    out = pl.pallas_call(
        _ln_body,
        grid=(b,),
        in_specs=[
            pl.BlockSpec((1, s, HID), lambda i: (i, 0, 0)),
            pl.BlockSpec((1, 1, s), lambda i: (i, 0, 0)),
            pl.BlockSpec((1, 1, s), lambda i: (i, 0, 0)),
            pl.BlockSpec((s, HID), lambda i: (0, 0)),
            pl.BlockSpec((2, HID), lambda i: (0, 0)),
            pl.BlockSpec((16, HID), lambda i: (0, 0)),
            pl.BlockSpec((1, HID), lambda i: (0, 0)),
            pl.BlockSpec((1, HID), lambda i: (0, 0)),
        ],
        out_specs=pl.BlockSpec((1, s, HID), lambda i: (i, 0, 0)),
        out_shape=jax.ShapeDtypeStruct((b, s, HID), jnp.float32),
        compiler_params=pltpu.CompilerParams(
            dimension_semantics=("arbitrary",)),
    )(gathered.reshape(b, s, HID), tt3, task3, pos_emb, tok_emb, task_emb,
      gamma.reshape(1, HID), beta.reshape(1, HID))
    return out


# speedup vs baseline: 5.0753x; 1.0125x over previous
import functools

import jax
import jax.numpy as jnp
from jax import lax
from jax.experimental import pallas as pl
from jax.experimental.pallas import tpu as pltpu
from jax.experimental.pallas import tpu_sc as plsc

HID = 128
EPS = 1e-12
CHUNK = 128


@functools.lru_cache(maxsize=None)
def _make_sc_gather(n_tokens: int):
    info = plsc.get_sparse_core_info()
    nc, ns = info.num_cores, info.num_subcores
    nw = nc * ns
    per_w = n_tokens // nw
    iters = per_w // CHUNK
    mesh = plsc.VectorSubcoreMesh(core_axis_name="c", subcore_axis_name="s")

    @functools.partial(
        pl.kernel,
        out_type=jax.ShapeDtypeStruct((n_tokens, HID), jnp.float32),
        mesh=mesh,
        scratch_types=[
            pltpu.VMEM((CHUNK,), jnp.int32),
            pltpu.VMEM((CHUNK, HID), jnp.float32),
            pltpu.SemaphoreType.DMA,
        ],
    )
    def gather(table_hbm, ids_hbm, out_hbm, idx_v, rows_v, sem):
        wid = lax.axis_index("s") * nc + lax.axis_index("c")
        base = wid * per_w

        def body(i, carry):
            off = base + i * CHUNK
            pltpu.sync_copy(ids_hbm.at[pl.ds(off, CHUNK)], idx_v)
            pltpu.async_copy(table_hbm.at[idx_v], rows_v, sem).wait()
            pltpu.sync_copy(rows_v, out_hbm.at[pl.ds(off, CHUNK)])
            return carry

        lax.fori_loop(0, iters, body, 0)

    return gather


def _ln_body(gath_ref, code_ref, pos_ref, ctab_ref, gamma_ref, beta_ref,
             out_ref):
    s = gath_ref.shape[1]
    code = code_ref[0, 0, :][:, None]
    oh = (code == lax.broadcasted_iota(jnp.int32, (s, 32), 1)
          ).astype(jnp.float32)
    e = (gath_ref[0] + pos_ref[...]
         + jnp.dot(oh, ctab_ref[...], preferred_element_type=jnp.float32))
    j = jnp.full((HID, HID), 1.0 / HID, dtype=jnp.float32)
    m1 = jnp.dot(e, j, preferred_element_type=jnp.float32)
    m2 = jnp.dot(e * e, j, preferred_element_type=jnp.float32)
    out = (e - m1) * lax.rsqrt(m2 - m1 * m1 + EPS) * gamma_ref[...] + beta_ref[...]
    out_ref[...] = out[None]


def kernel(input_ids, token_type_ids, task_type_ids, word_emb, pos_emb,
           tok_emb, task_emb, gamma, beta):
    b, s = input_ids.shape
    n = b * s
    ids = input_ids.reshape(n).astype(jnp.int32)
    gathered = _make_sc_gather(n)(word_emb, ids)

    code3 = (token_type_ids.astype(jnp.int32)
             + 2 * task_type_ids.astype(jnp.int32)).reshape(b, 1, s)
    ar = jnp.arange(32)
    ctab = tok_emb[ar % 2] + task_emb[ar // 2]
    out = pl.pallas_call(
        _ln_body,
        grid=(b,),
        in_specs=[
            pl.BlockSpec((1, s, HID), lambda i: (i, 0, 0)),
            pl.BlockSpec((1, 1, s), lambda i: (i, 0, 0)),
            pl.BlockSpec((s, HID), lambda i: (0, 0)),
            pl.BlockSpec((32, HID), lambda i: (0, 0)),
            pl.BlockSpec((1, HID), lambda i: (0, 0)),
            pl.BlockSpec((1, HID), lambda i: (0, 0)),
        ],
        out_specs=pl.BlockSpec((1, s, HID), lambda i: (i, 0, 0)),
        out_shape=jax.ShapeDtypeStruct((b, s, HID), jnp.float32),
        compiler_params=pltpu.CompilerParams(
            dimension_semantics=("arbitrary",)),
    )(gathered.reshape(b, s, HID), code3, pos_emb, ctab,
      gamma.reshape(1, HID), beta.reshape(1, HID))
    return out


# speedup vs baseline: 7.9289x; 1.5623x over previous
import functools

import jax
import jax.numpy as jnp
from jax import lax
from jax.experimental import pallas as pl
from jax.experimental.pallas import tpu as pltpu
from jax.experimental.pallas import tpu_sc as plsc

HID = 128
EPS = 1e-12
CHUNK = 128


@functools.lru_cache(maxsize=None)
def _make_sc_gather(n_tokens: int):
    info = plsc.get_sparse_core_info()
    nc, ns = info.num_cores, info.num_subcores
    nw = nc * ns
    per_w = n_tokens // nw
    iters = per_w // CHUNK
    mesh = plsc.VectorSubcoreMesh(core_axis_name="c", subcore_axis_name="s")

    @functools.partial(
        pl.kernel,
        out_type=jax.ShapeDtypeStruct((n_tokens, HID), jnp.float32),
        mesh=mesh,
        scratch_types=[
            pltpu.VMEM((CHUNK,), jnp.int32),
            pltpu.VMEM((CHUNK, HID), jnp.float32),
            pltpu.SemaphoreType.DMA,
        ],
    )
    def gather(table_hbm, ids_hbm, out_hbm, idx_v, rows_v, sem):
        wid = lax.axis_index("s") * nc + lax.axis_index("c")
        base = wid * per_w

        def body(i, carry):
            off = base + i * CHUNK
            pltpu.sync_copy(ids_hbm.at[pl.ds(off, CHUNK)], idx_v)
            pltpu.async_copy(table_hbm.at[idx_v], rows_v, sem).wait()
            pltpu.sync_copy(rows_v, out_hbm.at[pl.ds(off, CHUNK)])
            return carry

        lax.fori_loop(0, iters, body, 0)

    return gather


def _ln_body(gath_ref, code_ref, pos_ref, ctab_ref, gamma_ref, beta_ref,
             out_ref):
    r, s = gath_ref.shape[0], gath_ref.shape[1]
    j = jnp.full((HID, HID), 1.0 / HID, dtype=jnp.float32)
    for q in range(r):
        code = code_ref[q, 0, :][:, None]
        oh = (code == lax.broadcasted_iota(jnp.int32, (s, 32), 1)
              ).astype(jnp.float32)
        e = (gath_ref[q] + pos_ref[...]
             + jnp.dot(oh, ctab_ref[...], preferred_element_type=jnp.float32))
        m1 = jnp.dot(e, j, preferred_element_type=jnp.float32)
        m2 = jnp.dot(e * e, j, preferred_element_type=jnp.float32)
        out_ref[q] = ((e - m1) * lax.rsqrt(m2 - m1 * m1 + EPS)
                      * gamma_ref[...] + beta_ref[...])


def kernel(input_ids, token_type_ids, task_type_ids, word_emb, pos_emb,
           tok_emb, task_emb, gamma, beta):
    b, s = input_ids.shape
    n = b * s
    ids = input_ids.reshape(n).astype(jnp.int32)
    gathered = _make_sc_gather(n)(word_emb, ids)

    code3 = (token_type_ids.astype(jnp.int32)
             + 2 * task_type_ids.astype(jnp.int32)).reshape(b, 1, s)
    ar = jnp.arange(32)
    ctab = tok_emb[ar % 2] + task_emb[ar // 2]
    rows = 4
    out = pl.pallas_call(
        _ln_body,
        grid=(b // rows,),
        in_specs=[
            pl.BlockSpec((rows, s, HID), lambda i: (i, 0, 0)),
            pl.BlockSpec((rows, 1, s), lambda i: (i, 0, 0)),
            pl.BlockSpec((s, HID), lambda i: (0, 0)),
            pl.BlockSpec((32, HID), lambda i: (0, 0)),
            pl.BlockSpec((1, HID), lambda i: (0, 0)),
            pl.BlockSpec((1, HID), lambda i: (0, 0)),
        ],
        out_specs=pl.BlockSpec((rows, s, HID), lambda i: (i, 0, 0)),
        out_shape=jax.ShapeDtypeStruct((b, s, HID), jnp.float32),
        compiler_params=pltpu.CompilerParams(
            dimension_semantics=("arbitrary",)),
    )(gathered.reshape(b, s, HID), code3, pos_emb, ctab,
      gamma.reshape(1, HID), beta.reshape(1, HID))
    return out


# speedup vs baseline: 8.8786x; 1.1198x over previous
import functools

import jax
import jax.numpy as jnp
from jax import lax
from jax.experimental import pallas as pl
from jax.experimental.pallas import tpu as pltpu
from jax.experimental.pallas import tpu_sc as plsc

HID = 128
EPS = 1e-12
CHUNK = 128


@functools.lru_cache(maxsize=None)
def _make_sc_gather(n_tokens: int):
    info = plsc.get_sparse_core_info()
    nc, ns = info.num_cores, info.num_subcores
    nw = nc * ns
    per_w = n_tokens // nw
    iters = per_w // CHUNK
    mesh = plsc.VectorSubcoreMesh(core_axis_name="c", subcore_axis_name="s")

    @functools.partial(
        pl.kernel,
        out_type=jax.ShapeDtypeStruct((n_tokens, HID), jnp.float32),
        mesh=mesh,
        scratch_types=[
            pltpu.VMEM((CHUNK,), jnp.int32),
            pltpu.VMEM((CHUNK, HID), jnp.float32),
            pltpu.SemaphoreType.DMA,
        ],
    )
    def gather(table_hbm, ids_hbm, out_hbm, idx_v, rows_v, sem):
        wid = lax.axis_index("s") * nc + lax.axis_index("c")
        base = wid * per_w

        def body(i, carry):
            off = base + i * CHUNK
            pltpu.sync_copy(ids_hbm.at[pl.ds(off, CHUNK)], idx_v)
            pltpu.async_copy(table_hbm.at[idx_v], rows_v, sem).wait()
            pltpu.sync_copy(rows_v, out_hbm.at[pl.ds(off, CHUNK)])
            return carry

        lax.fori_loop(0, iters, body, 0)

    return gather


def _ln_body(gath_ref, code_ref, pos_ref, ctab_ref, gamma_ref, beta_ref,
             out_ref):
    r, s = gath_ref.shape[0], gath_ref.shape[1]
    j = jnp.full((HID, HID), 1.0 / HID, dtype=jnp.float32)
    for q in range(r):
        code = code_ref[q, 0, :][:, None]
        oh = (code == lax.broadcasted_iota(jnp.int32, (s, 32), 1)
              ).astype(jnp.float32)
        e = (gath_ref[q] + pos_ref[...]
             + jnp.dot(oh, ctab_ref[...], preferred_element_type=jnp.float32))
        m1 = jnp.dot(e, j, preferred_element_type=jnp.float32)
        m2 = jnp.dot(e * e, j, preferred_element_type=jnp.float32)
        out_ref[q] = ((e - m1) * lax.rsqrt(m2 - m1 * m1 + EPS)
                      * gamma_ref[...] + beta_ref[...])


def kernel(input_ids, token_type_ids, task_type_ids, word_emb, pos_emb,
           tok_emb, task_emb, gamma, beta):
    b, s = input_ids.shape
    n = b * s
    ids = input_ids.reshape(n).astype(jnp.int32)
    gathered = _make_sc_gather(n)(word_emb, ids)

    code3 = (token_type_ids.astype(jnp.int32)
             + 2 * task_type_ids.astype(jnp.int32)).reshape(b, 1, s)
    ar = jnp.arange(32)
    ctab = tok_emb[ar % 2] + task_emb[ar // 2]
    rows = 8
    out = pl.pallas_call(
        _ln_body,
        grid=(b // rows,),
        in_specs=[
            pl.BlockSpec((rows, s, HID), lambda i: (i, 0, 0)),
            pl.BlockSpec((rows, 1, s), lambda i: (i, 0, 0)),
            pl.BlockSpec((s, HID), lambda i: (0, 0)),
            pl.BlockSpec((32, HID), lambda i: (0, 0)),
            pl.BlockSpec((1, HID), lambda i: (0, 0)),
            pl.BlockSpec((1, HID), lambda i: (0, 0)),
        ],
        out_specs=pl.BlockSpec((rows, s, HID), lambda i: (i, 0, 0)),
        out_shape=jax.ShapeDtypeStruct((b, s, HID), jnp.float32),
        compiler_params=pltpu.CompilerParams(
            dimension_semantics=("arbitrary",)),
    )(gathered.reshape(b, s, HID), code3, pos_emb, ctab,
      gamma.reshape(1, HID), beta.reshape(1, HID))
    return out


# speedup vs baseline: 9.4700x; 1.0666x over previous
import functools

import jax
import jax.numpy as jnp
from jax import lax
from jax.experimental import pallas as pl
from jax.experimental.pallas import tpu as pltpu
from jax.experimental.pallas import tpu_sc as plsc

HID = 128
EPS = 1e-12
CHUNK = 128


@functools.lru_cache(maxsize=None)
def _make_sc_gather(n_tokens: int):
    info = plsc.get_sparse_core_info()
    nc, ns = info.num_cores, info.num_subcores
    nw = nc * ns
    per_w = n_tokens // nw
    iters = per_w // CHUNK
    mesh = plsc.VectorSubcoreMesh(core_axis_name="c", subcore_axis_name="s")

    @functools.partial(
        pl.kernel,
        out_type=jax.ShapeDtypeStruct((n_tokens, HID), jnp.float32),
        mesh=mesh,
        scratch_types=[
            pltpu.VMEM((CHUNK,), jnp.int32),
            pltpu.VMEM((CHUNK, HID), jnp.float32),
            pltpu.SemaphoreType.DMA,
        ],
    )
    def gather(table_hbm, ids_hbm, out_hbm, idx_v, rows_v, sem):
        wid = lax.axis_index("s") * nc + lax.axis_index("c")
        base = wid * per_w

        def body(i, carry):
            off = base + i * CHUNK
            pltpu.sync_copy(ids_hbm.at[pl.ds(off, CHUNK)], idx_v)
            pltpu.async_copy(table_hbm.at[idx_v], rows_v, sem).wait()
            pltpu.sync_copy(rows_v, out_hbm.at[pl.ds(off, CHUNK)])
            return carry

        lax.fori_loop(0, iters, body, 0)

    return gather


def _ln_body(gath_ref, code_ref, pos_ref, ctab_ref, gamma_ref, beta_ref,
             out_ref):
    r, s = gath_ref.shape[0], gath_ref.shape[1]
    j = jnp.full((HID, HID), 1.0 / HID, dtype=jnp.float32)
    for q in range(r):
        code = code_ref[q, 0, :][:, None]
        oh = (code == lax.broadcasted_iota(jnp.int32, (s, 32), 1)
              ).astype(jnp.float32)
        e = (gath_ref[q] + pos_ref[...]
             + jnp.dot(oh, ctab_ref[...], preferred_element_type=jnp.float32))
        m1 = jnp.dot(e, j, preferred_element_type=jnp.float32)
        m2 = jnp.dot(e * e, j, preferred_element_type=jnp.float32)
        out_ref[q] = ((e - m1) * lax.rsqrt(m2 - m1 * m1 + EPS)
                      * gamma_ref[...] + beta_ref[...])


def kernel(input_ids, token_type_ids, task_type_ids, word_emb, pos_emb,
           tok_emb, task_emb, gamma, beta):
    b, s = input_ids.shape
    n = b * s
    ids = input_ids.reshape(n).astype(jnp.int32)
    gathered = _make_sc_gather(n)(word_emb, ids)

    code3 = (token_type_ids.astype(jnp.int32)
             + 2 * task_type_ids.astype(jnp.int32)).reshape(b, 1, s)
    ar = jnp.arange(32)
    ctab = tok_emb[ar % 2] + task_emb[ar // 2]
    rows = 16
    out = pl.pallas_call(
        _ln_body,
        grid=(b // rows,),
        in_specs=[
            pl.BlockSpec((rows, s, HID), lambda i: (i, 0, 0)),
            pl.BlockSpec((rows, 1, s), lambda i: (i, 0, 0)),
            pl.BlockSpec((s, HID), lambda i: (0, 0)),
            pl.BlockSpec((32, HID), lambda i: (0, 0)),
            pl.BlockSpec((1, HID), lambda i: (0, 0)),
            pl.BlockSpec((1, HID), lambda i: (0, 0)),
        ],
        out_specs=pl.BlockSpec((rows, s, HID), lambda i: (i, 0, 0)),
        out_shape=jax.ShapeDtypeStruct((b, s, HID), jnp.float32),
        compiler_params=pltpu.CompilerParams(
            dimension_semantics=("arbitrary",)),
    )(gathered.reshape(b, s, HID), code3, pos_emb, ctab,
      gamma.reshape(1, HID), beta.reshape(1, HID))
    return out


# speedup vs baseline: 12.7859x; 1.3501x over previous
import functools

import jax
import jax.numpy as jnp
from jax import lax
from jax.experimental import pallas as pl
from jax.experimental.pallas import tpu as pltpu
from jax.experimental.pallas import tpu_sc as plsc

HID = 128
EPS = 1e-12
CHUNK = 128


@functools.lru_cache(maxsize=None)
def _make_sc_gather(n_tokens: int):
    info = plsc.get_sparse_core_info()
    nc, ns = info.num_cores, info.num_subcores
    nw = nc * ns
    per_w = n_tokens // nw
    iters = per_w // CHUNK
    mesh = plsc.VectorSubcoreMesh(core_axis_name="c", subcore_axis_name="s")

    npairs = iters // 2

    @functools.partial(
        pl.kernel,
        out_type=jax.ShapeDtypeStruct((n_tokens, HID), jnp.float32),
        mesh=mesh,
        scratch_types=[
            pltpu.VMEM((per_w,), jnp.int32),
            pltpu.VMEM((CHUNK, HID), jnp.float32),
            pltpu.VMEM((CHUNK, HID), jnp.float32),
            pltpu.SemaphoreType.DMA,
            pltpu.SemaphoreType.DMA,
        ],
    )
    def gather(table_hbm, ids_hbm, out_hbm, idx_v, rows0, rows1, sem0, sem1):
        wid = lax.axis_index("s") * nc + lax.axis_index("c")
        base = wid * per_w
        pltpu.sync_copy(ids_hbm.at[pl.ds(base, per_w)], idx_v)

        def g(i, rows, sem):
            return pltpu.make_async_copy(
                table_hbm.at[idx_v.at[pl.ds(i * CHUNK, CHUNK)]], rows, sem)

        g(0, rows0, sem0).start()

        def body(j, carry):
            i0 = 2 * j
            g(i0 + 1, rows1, sem1).start()
            g(i0, rows0, sem0).wait()
            pltpu.sync_copy(rows0, out_hbm.at[pl.ds(base + i0 * CHUNK, CHUNK)])

            @pl.when(j < npairs - 1)
            def _():
                g(i0 + 2, rows0, sem0).start()

            g(i0 + 1, rows1, sem1).wait()
            pltpu.sync_copy(
                rows1, out_hbm.at[pl.ds(base + (i0 + 1) * CHUNK, CHUNK)])
            return carry

        lax.fori_loop(0, npairs, body, 0)

    return gather


def _ln_body(gath_ref, code_ref, pos_ref, ctab_ref, gamma_ref, beta_ref,
             out_ref):
    r, s = gath_ref.shape[0], gath_ref.shape[1]
    j = jnp.full((HID, HID), 1.0 / HID, dtype=jnp.float32)
    for q in range(r):
        code = code_ref[q, 0, :][:, None]
        oh = (code == lax.broadcasted_iota(jnp.int32, (s, 32), 1)
              ).astype(jnp.float32)
        e = (gath_ref[q] + pos_ref[...]
             + jnp.dot(oh, ctab_ref[...], preferred_element_type=jnp.float32))
        m1 = jnp.dot(e, j, preferred_element_type=jnp.float32)
        m2 = jnp.dot(e * e, j, preferred_element_type=jnp.float32)
        out_ref[q] = ((e - m1) * lax.rsqrt(m2 - m1 * m1 + EPS)
                      * gamma_ref[...] + beta_ref[...])


def kernel(input_ids, token_type_ids, task_type_ids, word_emb, pos_emb,
           tok_emb, task_emb, gamma, beta):
    b, s = input_ids.shape
    n = b * s
    ids = input_ids.reshape(n).astype(jnp.int32)
    gathered = _make_sc_gather(n)(word_emb, ids)

    code3 = (token_type_ids.astype(jnp.int32)
             + 2 * task_type_ids.astype(jnp.int32)).reshape(b, 1, s)
    ar = jnp.arange(32)
    ctab = tok_emb[ar % 2] + task_emb[ar // 2]
    rows = 16
    out = pl.pallas_call(
        _ln_body,
        grid=(b // rows,),
        in_specs=[
            pl.BlockSpec((rows, s, HID), lambda i: (i, 0, 0)),
            pl.BlockSpec((rows, 1, s), lambda i: (i, 0, 0)),
            pl.BlockSpec((s, HID), lambda i: (0, 0)),
            pl.BlockSpec((32, HID), lambda i: (0, 0)),
            pl.BlockSpec((1, HID), lambda i: (0, 0)),
            pl.BlockSpec((1, HID), lambda i: (0, 0)),
        ],
        out_specs=pl.BlockSpec((rows, s, HID), lambda i: (i, 0, 0)),
        out_shape=jax.ShapeDtypeStruct((b, s, HID), jnp.float32),
        compiler_params=pltpu.CompilerParams(
            dimension_semantics=("arbitrary",)),
    )(gathered.reshape(b, s, HID), code3, pos_emb, ctab,
      gamma.reshape(1, HID), beta.reshape(1, HID))
    return out


# speedup vs baseline: 14.1243x; 1.1047x over previous
import functools

import jax
import jax.numpy as jnp
from jax import lax
from jax.experimental import pallas as pl
from jax.experimental.pallas import tpu as pltpu
from jax.experimental.pallas import tpu_sc as plsc

HID = 128
EPS = 1e-12
CHUNK = 128


@functools.lru_cache(maxsize=None)
def _make_sc_gather(n_tokens: int):
    info = plsc.get_sparse_core_info()
    nc, ns = info.num_cores, info.num_subcores
    nw = nc * ns
    per_w = n_tokens // nw
    iters = per_w // CHUNK
    mesh = plsc.VectorSubcoreMesh(core_axis_name="c", subcore_axis_name="s")

    npairs = iters // 2

    @functools.partial(
        pl.kernel,
        out_type=jax.ShapeDtypeStruct((n_tokens, HID), jnp.float32),
        mesh=mesh,
        scratch_types=[
            pltpu.VMEM((per_w,), jnp.int32),
            pltpu.VMEM((CHUNK, HID), jnp.float32),
            pltpu.VMEM((CHUNK, HID), jnp.float32),
            pltpu.SemaphoreType.DMA,
            pltpu.SemaphoreType.DMA,
        ],
    )
    def gather(table_hbm, ids_hbm, out_hbm, idx_v, rows0, rows1, sem0, sem1):
        wid = lax.axis_index("s") * nc + lax.axis_index("c")
        base = wid * per_w
        pltpu.sync_copy(ids_hbm.at[pl.ds(base, per_w)], idx_v)

        def g(i, rows, sem):
            return pltpu.make_async_copy(
                table_hbm.at[idx_v.at[pl.ds(i * CHUNK, CHUNK)]], rows, sem)

        g(0, rows0, sem0).start()

        def body(j, carry):
            i0 = 2 * j
            g(i0 + 1, rows1, sem1).start()
            g(i0, rows0, sem0).wait()
            pltpu.sync_copy(rows0, out_hbm.at[pl.ds(base + i0 * CHUNK, CHUNK)])

            @pl.when(j < npairs - 1)
            def _():
                g(i0 + 2, rows0, sem0).start()

            g(i0 + 1, rows1, sem1).wait()
            pltpu.sync_copy(
                rows1, out_hbm.at[pl.ds(base + (i0 + 1) * CHUNK, CHUNK)])
            return carry

        lax.fori_loop(0, npairs, body, 0)

    return gather


def _ln_body(gath_ref, code_ref, pos_ref, ctab_ref, gamma_ref, beta_ref,
             out_ref):
    r, s = gath_ref.shape[0], gath_ref.shape[1]
    j = jnp.full((HID, HID), 1.0 / HID, dtype=jnp.float32)
    for q in range(r):
        code = code_ref[q, 0, :][:, None]
        oh = (code == lax.broadcasted_iota(jnp.int32, (s, 32), 1)
              ).astype(jnp.float32)
        e = (gath_ref[q] + pos_ref[...]
             + jnp.dot(oh, ctab_ref[...], preferred_element_type=jnp.float32))
        m1 = jnp.dot(e, j, preferred_element_type=jnp.float32)
        m2 = jnp.dot(e * e, j, preferred_element_type=jnp.float32)
        out_ref[q] = ((e - m1) * lax.rsqrt(m2 - m1 * m1 + EPS)
                      * gamma_ref[...] + beta_ref[...])


def _ln_body_chain(dst_ref, gath_ref, code_ref, pos_ref, ctab_ref, gamma_ref,
                   beta_ref, out_ref):
    del dst_ref
    _ln_body(gath_ref, code_ref, pos_ref, ctab_ref, gamma_ref, beta_ref,
             out_ref)


def kernel(input_ids, token_type_ids, task_type_ids, word_emb, pos_emb,
           tok_emb, task_emb, gamma, beta):
    b, s = input_ids.shape
    nsl = 4
    rows = 16
    bsl = b // nsl
    grid_k = bsl // rows
    ids = input_ids.reshape(nsl, bsl * s).astype(jnp.int32)
    code4 = (token_type_ids.astype(jnp.int32)
             + 2 * task_type_ids.astype(jnp.int32)).reshape(nsl, bsl, 1, s)
    ar = jnp.arange(32)
    ctab = tok_emb[ar % 2] + task_emb[ar // 2]
    gamma2 = gamma.reshape(1, HID)
    beta2 = beta.reshape(1, HID)

    sc_gather = _make_sc_gather(bsl * s)
    gaths = [sc_gather(word_emb, ids[k]) for k in range(nsl)]

    in_specs = [
        pl.BlockSpec((rows, s, HID), lambda i: (i, 0, 0)),
        pl.BlockSpec((rows, 1, s), lambda i: (i, 0, 0)),
        pl.BlockSpec((s, HID), lambda i: (0, 0)),
        pl.BlockSpec((32, HID), lambda i: (0, 0)),
        pl.BlockSpec((1, HID), lambda i: (0, 0)),
        pl.BlockSpec((1, HID), lambda i: (0, 0)),
    ]
    out_shape = jax.ShapeDtypeStruct((b, s, HID), jnp.float32)
    cparams = pltpu.CompilerParams(dimension_semantics=("arbitrary",))

    out = None
    for k in range(nsl):
        args = (gaths[k].reshape(bsl, s, HID), code4[k], pos_emb, ctab,
                gamma2, beta2)
        out_spec = pl.BlockSpec(
            (rows, s, HID), lambda i, kk=k: (kk * grid_k + i, 0, 0))
        if k == 0:
            out = pl.pallas_call(
                _ln_body, grid=(grid_k,), in_specs=in_specs,
                out_specs=out_spec, out_shape=out_shape,
                compiler_params=cparams)(*args)
        else:
            out = pl.pallas_call(
                _ln_body_chain, grid=(grid_k,),
                in_specs=[pl.BlockSpec(memory_space=pl.ANY)] + in_specs,
                out_specs=out_spec, out_shape=out_shape,
                input_output_aliases={0: 0},
                compiler_params=cparams)(out, *args)
    return out
